# baseline (speedup 1.0000x reference)
import functools

import jax
import jax.numpy as jnp
from jax import lax
from jax.experimental import pallas as pl
from jax.experimental.pallas import tpu as pltpu
from jax.experimental.pallas import tpu_sc as plsc

NN = 10000
EE = 320000
GG = 64
HID = 128
NCLS = 3
TAB = 512
ACCW = 144
NWORK = 32
EPW = EE // NWORK
CHB = 50
NCH = EPW // CHB
NRCH = NN // CHB


def _f32(x):
    return x.astype(jnp.float32)


def _tc0_body(x_ref, ea0_ref, ea1_ref, aembW_ref, alinW_ref, alinb_ref,
              eembW_ref, elinW_ref, elinb_ref, Wl_ref, bl_ref, Wr_ref,
              br_ref, We_ref,
              xl_out, xr_out, eptab_out, tab_out, eid_out):
    x0 = x_ref[:, 0:1]
    oh = _f32(x0 == lax.broadcasted_iota(jnp.int32, (NN, 120), 1))
    embp = jnp.dot(oh, aembW_ref[...], preferred_element_type=jnp.float32, precision=lax.Precision.HIGHEST)
    sh = 7 - lax.broadcasted_iota(jnp.int32, (NN, 8), 1)
    parts = []
    for i in range(7):
        b = x_ref[:, i + 1:i + 2]
        parts.append(_f32((b >> sh) & 1))
    bits = jnp.concatenate(parts, axis=1)
    linp = (jnp.dot(bits, alinW_ref[...], preferred_element_type=jnp.float32)
            + alinb_ref[...])
    h = jnp.concatenate([embp, linp], axis=1)
    xl_out[...] = (jnp.dot(h, Wl_ref[...], preferred_element_type=jnp.float32)
                   + bl_ref[...])
    xr_out[...] = (jnp.dot(h, Wr_ref[...], preferred_element_type=jnp.float32)
                   + br_ref[...])
    ti = lax.broadcasted_iota(jnp.int32, (TAB, 1), 0)
    a0 = ti // 22
    a1 = ti % 22
    oh0 = _f32(a0 == lax.broadcasted_iota(jnp.int32, (TAB, 22), 1))
    embe = jnp.dot(oh0, eembW_ref[...], preferred_element_type=jnp.float32, precision=lax.Precision.HIGHEST)
    sh8 = 7 - lax.broadcasted_iota(jnp.int32, (TAB, 8), 1)
    bits8 = _f32((a1 >> sh8) & 1)
    line = (jnp.dot(bits8, elinW_ref[...], preferred_element_type=jnp.float32)
            + elinb_ref[...])
    tab = jnp.concatenate([embe, line], axis=1)
    tab_out[...] = tab
    eptab_out[...] = jnp.dot(tab, We_ref[...],
                             preferred_element_type=jnp.float32)
    eid_out[...] = ea0_ref[...] * 22 + ea1_ref[...]


def _tc0(x, ea0, ea1, aembW, alinW, alinb, eembW, elinW, elinb,
         Wl, bl, Wr, br, We):
    return pl.pallas_call(
        _tc0_body,
        out_shape=(
            jax.ShapeDtypeStruct((NN, HID), jnp.float32),
            jax.ShapeDtypeStruct((NN, HID), jnp.float32),
            jax.ShapeDtypeStruct((TAB, HID), jnp.float32),
            jax.ShapeDtypeStruct((TAB, 32), jnp.float32),
            jax.ShapeDtypeStruct((EE // 128, 128), jnp.int32),
        ),
    )(x, ea0, ea1, aembW, alinW, alinb, eembW, elinW, elinb,
      Wl, bl, Wr, br, We)


def _make_sc_edge(H):
    mesh = plsc.VectorSubcoreMesh(core_axis_name="c", subcore_axis_name="s")
    npr = 16 // H
    DR = NN // npr
    DFULL = DR // CHB
    DREM = DR % CHB
    shift = 2 if H == 4 else 4

    @functools.partial(
        pl.kernel,
        out_type=(jax.ShapeDtypeStruct((2, NN, HID), jnp.float32),
                  jax.ShapeDtypeStruct((2, DR, 16), jnp.float32)),
        mesh=mesh,
        compiler_params=pltpu.CompilerParams(needs_layout_passes=False,
                                             use_tc_tiling_on_sc=False),
        scratch_types=[
            [pltpu.VMEM((CHB,), jnp.int32)] * 4,
            [pltpu.VMEM((CHB,), jnp.int32)] * 4,
            [pltpu.VMEM((CHB,), jnp.int32)] * 4,
            [pltpu.VMEM((CHB,), jnp.int32)] * 2,
            pltpu.VMEM((2, CHB, HID), jnp.float32),
            pltpu.VMEM((2, CHB, HID), jnp.float32),
            pltpu.VMEM((2, CHB, HID), jnp.float32),
            pltpu.VMEM((2, CHB, 16), jnp.float32),
            pltpu.VMEM((HID,), jnp.float32),
            pltpu.VMEM_SHARED((NN, HID), jnp.float32),
            pltpu.VMEM_SHARED((DR, 16), jnp.float32),
            pltpu.SemaphoreType.DMA((4,)),
            pltpu.SemaphoreType.DMA((6,)),
            pltpu.SemaphoreType.DMA((4,)),
        ],
    )
    def sc_edge(xl_hbm, xr_hbm, ep_hbm, src_hbm, dst_hbm, eid_hbm, att_hbm,
                out_hbm, dout_hbm, srcb, dstb, eidb, didx_v, xl_r, xr_r,
                ep_r, den_st, att_v, acc_sh, den_sh, isems, gsems, ssems):
        cid = lax.axis_index("c")
        sid = lax.axis_index("s")
        wid = cid * 16 + sid
        zero = jnp.zeros((16,), jnp.float32)
        lane = lax.iota(jnp.int32, 16)

        def zrow(r, _):
            for j in range(HID // 16):
                xl_r[0, r, pl.ds(j * 16, 16)] = zero
            return 0
        lax.fori_loop(0, CHB, zrow, 0, unroll=False)

        def zrow2(r, _):
            den_st[0, r, pl.ds(0, 16)] = zero
            return 0
        lax.fori_loop(0, CHB, zrow2, 0, unroll=False)
        for i in range((NRCH + 15) // 16):
            k = i * 16 + sid

            @pl.when(k < NRCH)
            def _():
                pltpu.sync_copy(xl_r.at[0], acc_sh.at[pl.ds(k * CHB, CHB)])
        for i in range((DFULL + 16) // 16):
            k = i * 16 + sid

            @pl.when(k < DFULL)
            def _():
                pltpu.sync_copy(den_st.at[0], den_sh.at[pl.ds(k * CHB, CHB)])
            if DREM:
                @pl.when(k == DFULL)
                def _():
                    pltpu.sync_copy(den_st.at[0, pl.ds(0, DREM)],
                                    den_sh.at[pl.ds(DFULL * CHB, DREM)])
        plsc.subcore_barrier()

        pltpu.sync_copy(att_hbm, att_v)
        attv = [att_v[pl.ds(j * 16, 16)] for j in range(8)]

        def start_idx(k, r):
            pltpu.async_copy(src_hbm.at[wid, k], srcb[r], isems.at[r])
            pltpu.async_copy(dst_hbm.at[wid, k], dstb[r], isems.at[r])
            pltpu.async_copy(eid_hbm.at[wid, k], eidb[r], isems.at[r])

        def wait_idx(r):
            for _ in range(3):
                pltpu.make_async_copy(src_hbm.at[wid, 0], srcb[r],
                                      isems.at[r]).wait()

        def start_gather(r, p):
            pltpu.async_copy(xl_hbm.at[srcb[r]], xl_r.at[p], gsems.at[3 * p])
            pltpu.async_copy(xr_hbm.at[dstb[r]], xr_r.at[p],
                             gsems.at[3 * p + 1])
            pltpu.async_copy(ep_hbm.at[eidb[r]], ep_r.at[p],
                             gsems.at[3 * p + 2])

        def wait_gather(p):
            for t, ref in ((0, xl_r), (1, xr_r), (2, ep_r)):
                pltpu.make_async_copy(xl_hbm.at[srcb[0]], ref.at[p],
                                      gsems.at[3 * p + t]).wait()

        def start_scatter(r, p):
            pltpu.async_copy(xl_r.at[p], acc_sh.at[dstb[r]],
                             ssems.at[2 * p], add=True)
            pltpu.async_copy(den_st.at[p], den_sh.at[didx_v[p]],
                             ssems.at[2 * p + 1], add=True)

        def wait_scatter(p):
            pltpu.make_async_copy(xl_r.at[p], acc_sh.at[dstb[0]],
                                  ssems.at[2 * p]).wait()
            pltpu.make_async_copy(den_st.at[p], den_sh.at[didx_v[0]],
                                  ssems.at[2 * p + 1]).wait()

        def compute(r, p):
            for off in (0, 16, 32, CHB - 16):
                dv = dstb[r][pl.ds(off, 16)]
                didx_v[p][pl.ds(off, 16)] = dv >> shift

            def edge(e, _):
                egrp = jnp.minimum(e - (e & 15), CHB - 16)
                dvec = dstb[r][pl.ds(egrp, 16)]
                dsc = jnp.sum(jnp.where(lane == (e - egrp), dvec, 0))
                xs = []
                ts = []
                for j in range(8):
                    xlj = xl_r[p, e, pl.ds(j * 16, 16)]
                    m = (xlj + xr_r[p, e, pl.ds(j * 16, 16)]
                         + ep_r[p, e, pl.ds(j * 16, 16)])
                    act = jnp.maximum(m, 0.2 * m)
                    xs.append(xlj)
                    ts.append(act * attv[j])
                if H == 4:
                    l0 = (dsc & 3) * 4
                    den = zero
                    aes = []
                    for hh in range(4):
                        alpha = jnp.minimum(
                            jnp.sum(ts[2 * hh] + ts[2 * hh + 1]), 75.0)
                        aev = jnp.exp(jnp.full((16,), alpha, jnp.float32))
                        aes.append(aev)
                        den = jnp.where(lane == l0 + hh, aev, den)
                    for j in range(8):
                        xl_r[p, e, pl.ds(j * 16, 16)] = xs[j] * aes[j // 2]
                    den_st[p, e, pl.ds(0, 16)] = den
                else:
                    l0 = dsc & 15
                    s = ts[0]
                    for j in range(1, 8):
                        s = s + ts[j]
                    alpha = jnp.minimum(jnp.sum(s), 75.0)
                    aev = jnp.exp(jnp.full((16,), alpha, jnp.float32))
                    for j in range(8):
                        xl_r[p, e, pl.ds(j * 16, 16)] = xs[j] * aev
                    den_st[p, e, pl.ds(0, 16)] = jnp.where(lane == l0, aev,
                                                           zero)
                return 0

            lax.fori_loop(0, CHB, edge, 0, unroll=5)

        start_idx(0, 0)
        start_idx(1, 1)
        start_idx(2, 2)
        wait_idx(0)
        start_gather(0, 0)

        def group(g, _):
            for c in range(4):
                k = 4 * g + c
                r = c
                p = c % 2

                @pl.when(k >= 1)
                def _():
                    wait_scatter(1 - p)

                @pl.when(k + 1 < NCH)
                def _():
                    wait_idx((c + 1) % 4)
                    start_gather((c + 1) % 4, 1 - p)
                wait_gather(p)
                compute(r, p)
                start_scatter(r, p)

                @pl.when(k + 3 < NCH)
                def _():
                    start_idx(k + 3, (c + 3) % 4)
            return 0

        lax.fori_loop(0, NCH // 4, group, 0, unroll=False)
        wait_scatter(1)
        plsc.subcore_barrier()

        for i in range((NRCH + 15) // 16):
            k = i * 16 + sid

            @pl.when(k < NRCH)
            def _():
                pltpu.sync_copy(acc_sh.at[pl.ds(k * CHB, CHB)], xl_r.at[0])
                pltpu.sync_copy(xl_r.at[0],
                                out_hbm.at[cid, pl.ds(k * CHB, CHB)])
        for i in range((DFULL + 16) // 16):
            k = i * 16 + sid

            @pl.when(k < DFULL)
            def _():
                pltpu.sync_copy(den_sh.at[pl.ds(k * CHB, CHB)], den_st.at[0])
                pltpu.sync_copy(den_st.at[0],
                                dout_hbm.at[cid, pl.ds(k * CHB, CHB)])
            if DREM:
                @pl.when(k == DFULL)
                def _():
                    pltpu.sync_copy(den_sh.at[pl.ds(DFULL * CHB, DREM)],
                                    den_st.at[0, pl.ds(0, DREM)])
                    pltpu.sync_copy(den_st.at[0, pl.ds(0, DREM)],
                                    dout_hbm.at[cid, pl.ds(DFULL * CHB, DREM)])

    return sc_edge


_sc_edge_h4 = _make_sc_edge(4)
_sc_edge_h1 = _make_sc_edge(1)


def _combine_bn(acc_ref, den_ref, gamma_ref, beta_ref, cbias_ref, H, final):
    num = acc_ref[0] + acc_ref[1]
    den = den_ref[0] + den_ref[1]
    row = lax.broadcasted_iota(jnp.int32, (H, HID), 0)
    col = lax.broadcasted_iota(jnp.int32, (H, HID), 1)
    if H == 4:
        R = _f32(row == col // 32)
    else:
        R = jnp.ones((1, HID), jnp.float32) + 0.0 * _f32(row + col)
    denf = jnp.dot(den, R, preferred_element_type=jnp.float32, precision=lax.Precision.HIGHEST)
    o = num / (denf + 1e-16) + cbias_ref[...]
    mu = jnp.sum(o, axis=0, keepdims=True) * (1.0 / NN)
    d = o - mu
    var = jnp.sum(d * d, axis=0, keepdims=True) * (1.0 / NN)
    o = d * lax.rsqrt(var + 1e-5) * gamma_ref[...] + beta_ref[...]
    if not final:
        o = jnp.where(o >= 0.0, o, 0.01 * o)
    return o


def _make_tc_mid(H):
    def body(acc_ref, den_ref, gamma_ref, beta_ref, cbias_ref, Wl_ref, bl_ref,
             Wr_ref, br_ref, We_ref, tab_ref,
             xl_out, xr_out, eptab_out):
        o = _combine_bn(acc_ref, den_ref, gamma_ref, beta_ref, cbias_ref, H,
                        False)
        xl_out[...] = (jnp.dot(o, Wl_ref[...],
                               preferred_element_type=jnp.float32)
                       + bl_ref[...])
        xr_out[...] = (jnp.dot(o, Wr_ref[...],
                               preferred_element_type=jnp.float32)
                       + br_ref[...])
        eptab_out[...] = jnp.dot(tab_ref[...], We_ref[...],
                                 preferred_element_type=jnp.float32)

    def run(acc, den, gamma, beta, cbias, Wl, bl, Wr, br, We, tab):
        return pl.pallas_call(
            body,
            out_shape=(
                jax.ShapeDtypeStruct((NN, HID), jnp.float32),
                jax.ShapeDtypeStruct((NN, HID), jnp.float32),
                jax.ShapeDtypeStruct((TAB, HID), jnp.float32),
            ),
        )(acc, den, gamma, beta, cbias, Wl, bl, Wr, br, We, tab)

    return run


_tc_mid_h4 = _make_tc_mid(4)
_tc_mid_h1 = _make_tc_mid(1)


def _tc_final_body(acc_ref, den_ref, gamma_ref, beta_ref, cbias_ref,
                   batch_ref, W1_ref, b1_ref, W2_ref, b2_ref, W3_ref, b3_ref,
                   W4_ref, b4_ref, out_ref):
    o = _combine_bn(acc_ref, den_ref, gamma_ref, beta_ref, cbias_ref, 1, True)
    oh = _f32(batch_ref[...] == lax.broadcasted_iota(jnp.int32, (NN, GG), 1))
    dnum = (((0,), (0,)), ((), ()))
    sums = lax.dot_general(oh, o, dnum,
                           preferred_element_type=jnp.float32, precision=lax.Precision.HIGHEST)
    cnt = jnp.sum(oh, axis=0, keepdims=True)
    g = sums / jnp.maximum(cnt.reshape(GG, 1), 1.0)
    g = jnp.maximum(jnp.dot(g, W1_ref[...],
                            preferred_element_type=jnp.float32) + b1_ref[...],
                    0.0)
    g = jnp.maximum(jnp.dot(g, W2_ref[...],
                            preferred_element_type=jnp.float32) + b2_ref[...],
                    0.0)
    g = jnp.maximum(jnp.dot(g, W3_ref[...],
                            preferred_element_type=jnp.float32) + b3_ref[...],
                    0.0)
    out_ref[...] = (jnp.dot(g, W4_ref[...],
                            preferred_element_type=jnp.float32) + b4_ref[...])


def _tc_final(acc, den, gamma, beta, cbias, batch, W1, b1, W2, b2, W3, b3,
              W4, b4):
    return pl.pallas_call(
        _tc_final_body,
        out_shape=jax.ShapeDtypeStruct((GG, NCLS), jnp.float32),
    )(acc, den, gamma, beta, cbias, batch, W1, b1, W2, b2, W3, b3, W4, b4)


def kernel(x, edge_index, edge_attr, batch, atom_emb_W, atom_lin_W,
           atom_lin_b, edge_emb_W, edge_lin_W, edge_lin_b, lin_l_W, lin_l_b,
           lin_r_W, lin_r_b, lin_edge_W, att, conv_bias, bn_gamma, bn_beta,
           W1, b1, W2, b2, W3, b3, W4, b4):
    src = edge_index[0].reshape(NWORK, NCH, CHB)
    dst = edge_index[1].reshape(NWORK, NCH, CHB)
    ea0 = edge_attr[:, 0].reshape(EE // 128, 128)
    ea1 = edge_attr[:, 1].reshape(EE // 128, 128)

    xl, xr, eptab, tab, eid = _tc0(
        x, ea0, ea1, atom_emb_W, atom_lin_W, atom_lin_b.reshape(1, -1),
        edge_emb_W, edge_lin_W, edge_lin_b.reshape(1, -1),
        lin_l_W[0], lin_l_b[0].reshape(1, -1),
        lin_r_W[0], lin_r_b[0].reshape(1, -1), lin_edge_W[0])
    eid = eid.reshape(NWORK, NCH, CHB)

    acc, dacc = _sc_edge_h4(xl, xr, eptab, src, dst, eid, att[0])
    xl, xr, eptab = _tc_mid_h4(
        acc, dacc.reshape(2, NN, 4),
        bn_gamma[0].reshape(1, -1), bn_beta[0].reshape(1, -1),
        conv_bias[0].reshape(1, -1),
        lin_l_W[1], lin_l_b[1].reshape(1, -1),
        lin_r_W[1], lin_r_b[1].reshape(1, -1), lin_edge_W[1], tab)

    acc, dacc = _sc_edge_h1(xl, xr, eptab, src, dst, eid, att[1])
    xl, xr, eptab = _tc_mid_h1(
        acc, dacc.reshape(2, NN, 1),
        bn_gamma[1].reshape(1, -1), bn_beta[1].reshape(1, -1),
        conv_bias[1].reshape(1, -1),
        lin_l_W[2], lin_l_b[2].reshape(1, -1),
        lin_r_W[2], lin_r_b[2].reshape(1, -1), lin_edge_W[2], tab)

    acc, dacc = _sc_edge_h1(xl, xr, eptab, src, dst, eid, att[2])
    out = _tc_final(
        acc, dacc.reshape(2, NN, 1),
        bn_gamma[2].reshape(1, -1), bn_beta[2].reshape(1, -1),
        conv_bias[2].reshape(1, -1), batch.reshape(NN, 1),
        W1, b1.reshape(1, -1), W2, b2.reshape(1, -1),
        W3, b3.reshape(1, -1), W4, b4.reshape(1, -1))
    return out


# speedup vs baseline: 1.7422x; 1.7422x over previous
import functools

import jax
import jax.numpy as jnp
from jax import lax
from jax.experimental import pallas as pl
from jax.experimental.pallas import tpu as pltpu
from jax.experimental.pallas import tpu_sc as plsc

NN = 10000
EE = 320000
GG = 64
HID = 128
NCLS = 3
TAB = 512
ACCW = 144
NWORK = 32
EPW = EE // NWORK
CHB = 50
NCH = EPW // CHB
NRCH = NN // CHB


def _f32(x):
    return x.astype(jnp.float32)


def _tc0_body(x_ref, ea0_ref, ea1_ref, aembW_ref, alinW_ref, alinb_ref,
              eembW_ref, elinW_ref, elinb_ref, Wl_ref, bl_ref, Wr_ref,
              br_ref, We_ref,
              xl_out, xr_out, eptab_out, tab_out, eid_out):
    x0 = x_ref[:, 0:1]
    oh = _f32(x0 == lax.broadcasted_iota(jnp.int32, (NN, 120), 1))
    embp = jnp.dot(oh, aembW_ref[...], preferred_element_type=jnp.float32, precision=lax.Precision.HIGHEST)
    sh = 7 - lax.broadcasted_iota(jnp.int32, (NN, 8), 1)
    parts = []
    for i in range(7):
        b = x_ref[:, i + 1:i + 2]
        parts.append(_f32((b >> sh) & 1))
    bits = jnp.concatenate(parts, axis=1)
    linp = (jnp.dot(bits, alinW_ref[...], preferred_element_type=jnp.float32)
            + alinb_ref[...])
    h = jnp.concatenate([embp, linp], axis=1)
    xl_out[...] = (jnp.dot(h, Wl_ref[...], preferred_element_type=jnp.float32)
                   + bl_ref[...])
    xr_out[...] = (jnp.dot(h, Wr_ref[...], preferred_element_type=jnp.float32)
                   + br_ref[...])
    ti = lax.broadcasted_iota(jnp.int32, (TAB, 1), 0)
    a0 = ti // 22
    a1 = ti % 22
    oh0 = _f32(a0 == lax.broadcasted_iota(jnp.int32, (TAB, 22), 1))
    embe = jnp.dot(oh0, eembW_ref[...], preferred_element_type=jnp.float32, precision=lax.Precision.HIGHEST)
    sh8 = 7 - lax.broadcasted_iota(jnp.int32, (TAB, 8), 1)
    bits8 = _f32((a1 >> sh8) & 1)
    line = (jnp.dot(bits8, elinW_ref[...], preferred_element_type=jnp.float32)
            + elinb_ref[...])
    tab = jnp.concatenate([embe, line], axis=1)
    tab_out[...] = tab
    eptab_out[...] = jnp.dot(tab, We_ref[...],
                             preferred_element_type=jnp.float32)
    eid_out[...] = ea0_ref[...] * 22 + ea1_ref[...]


def _tc0(x, ea0, ea1, aembW, alinW, alinb, eembW, elinW, elinb,
         Wl, bl, Wr, br, We):
    return pl.pallas_call(
        _tc0_body,
        out_shape=(
            jax.ShapeDtypeStruct((NN, HID), jnp.float32),
            jax.ShapeDtypeStruct((NN, HID), jnp.float32),
            jax.ShapeDtypeStruct((TAB, HID), jnp.float32),
            jax.ShapeDtypeStruct((TAB, 32), jnp.float32),
            jax.ShapeDtypeStruct((EE // 128, 128), jnp.int32),
        ),
    )(x, ea0, ea1, aembW, alinW, alinb, eembW, elinW, elinb,
      Wl, bl, Wr, br, We)


def _make_sc_edge(H):
    mesh = plsc.VectorSubcoreMesh(core_axis_name="c", subcore_axis_name="s")
    npr = 16 // H
    DR = NN // npr
    DFULL = DR // CHB
    DREM = DR % CHB
    shift = 2 if H == 4 else 4

    @functools.partial(
        pl.kernel,
        out_type=(jax.ShapeDtypeStruct((2, NN, HID), jnp.float32),
                  jax.ShapeDtypeStruct((2, DR, 16), jnp.float32)),
        mesh=mesh,
        compiler_params=pltpu.CompilerParams(needs_layout_passes=False,
                                             use_tc_tiling_on_sc=False),
        scratch_types=[
            [pltpu.VMEM((CHB,), jnp.int32)] * 4,
            [pltpu.VMEM((CHB,), jnp.int32)] * 4,
            [pltpu.VMEM((CHB,), jnp.int32)] * 4,
            [pltpu.VMEM((CHB,), jnp.int32)] * 2,
            pltpu.VMEM((2, CHB, HID), jnp.float32),
            pltpu.VMEM((2, CHB, HID), jnp.float32),
            pltpu.VMEM((2, CHB, HID), jnp.float32),
            pltpu.VMEM((2, CHB, 16), jnp.float32),
            pltpu.VMEM((HID,), jnp.float32),
            pltpu.VMEM_SHARED((NN, HID), jnp.float32),
            pltpu.VMEM_SHARED((DR, 16), jnp.float32),
            pltpu.SemaphoreType.DMA((4,)),
            pltpu.SemaphoreType.DMA((6,)),
            pltpu.SemaphoreType.DMA((4,)),
        ],
    )
    def sc_edge(xl_hbm, xr_hbm, ep_hbm, src_hbm, dst_hbm, eid_hbm, att_hbm,
                out_hbm, dout_hbm, srcb, dstb, eidb, didx_v, xl_r, xr_r,
                ep_r, den_st, att_v, acc_sh, den_sh, isems, gsems, ssems):
        cid = lax.axis_index("c")
        sid = lax.axis_index("s")
        wid = cid * 16 + sid
        zero = jnp.zeros((16,), jnp.float32)
        lane = lax.iota(jnp.int32, 16)

        def zrow(r, _):
            for j in range(HID // 16):
                xl_r[0, r, pl.ds(j * 16, 16)] = zero
            return 0
        lax.fori_loop(0, CHB, zrow, 0, unroll=False)

        def zrow2(r, _):
            den_st[0, r, pl.ds(0, 16)] = zero
            return 0
        lax.fori_loop(0, CHB, zrow2, 0, unroll=False)
        for i in range((NRCH + 15) // 16):
            k = i * 16 + sid

            @pl.when(k < NRCH)
            def _():
                pltpu.sync_copy(xl_r.at[0], acc_sh.at[pl.ds(k * CHB, CHB)])
        for i in range((DFULL + 16) // 16):
            k = i * 16 + sid

            @pl.when(k < DFULL)
            def _():
                pltpu.sync_copy(den_st.at[0], den_sh.at[pl.ds(k * CHB, CHB)])
            if DREM:
                @pl.when(k == DFULL)
                def _():
                    pltpu.sync_copy(den_st.at[0, pl.ds(0, DREM)],
                                    den_sh.at[pl.ds(DFULL * CHB, DREM)])
        plsc.subcore_barrier()

        pltpu.sync_copy(att_hbm, att_v)
        attv = [att_v[pl.ds(j * 16, 16)] for j in range(8)]

        def start_idx(k, r):
            pltpu.async_copy(src_hbm.at[wid, k], srcb[r], isems.at[r])
            pltpu.async_copy(dst_hbm.at[wid, k], dstb[r], isems.at[r])
            pltpu.async_copy(eid_hbm.at[wid, k], eidb[r], isems.at[r])

        def wait_idx(r):
            for _ in range(3):
                pltpu.make_async_copy(src_hbm.at[wid, 0], srcb[r],
                                      isems.at[r]).wait()

        def start_gather(r, p):
            pltpu.async_copy(xl_hbm.at[srcb[r]], xl_r.at[p], gsems.at[3 * p])
            pltpu.async_copy(xr_hbm.at[dstb[r]], xr_r.at[p],
                             gsems.at[3 * p + 1])
            pltpu.async_copy(ep_hbm.at[eidb[r]], ep_r.at[p],
                             gsems.at[3 * p + 2])

        def wait_gather(p):
            for t, ref in ((0, xl_r), (1, xr_r), (2, ep_r)):
                pltpu.make_async_copy(xl_hbm.at[srcb[0]], ref.at[p],
                                      gsems.at[3 * p + t]).wait()

        def start_scatter(r, p):
            pltpu.async_copy(xl_r.at[p], acc_sh.at[dstb[r]],
                             ssems.at[2 * p], add=True)
            pltpu.async_copy(den_st.at[p], den_sh.at[didx_v[p]],
                             ssems.at[2 * p + 1], add=True)

        def wait_scatter(p):
            pltpu.make_async_copy(xl_r.at[p], acc_sh.at[dstb[0]],
                                  ssems.at[2 * p]).wait()
            pltpu.make_async_copy(den_st.at[p], den_sh.at[didx_v[0]],
                                  ssems.at[2 * p + 1]).wait()

        def compute(r, p):
            for off in (0, 16, 32, CHB - 16):
                dv = dstb[r][pl.ds(off, 16)]
                didx_v[p][pl.ds(off, 16)] = dv >> shift

            @plsc.parallel_loop(0, CHB, 1, unroll=2)
            def edge(e):
                egrp = jnp.minimum(e - (e & 15), CHB - 16)
                dvec = dstb[r][pl.ds(egrp, 16)]
                dsc = jnp.sum(jnp.where(lane == (e - egrp), dvec, 0))
                xs = []
                ts = []
                for j in range(8):
                    xlj = xl_r[p, e, pl.ds(j * 16, 16)]
                    m = (xlj + xr_r[p, e, pl.ds(j * 16, 16)]
                         + ep_r[p, e, pl.ds(j * 16, 16)])
                    act = jnp.maximum(m, 0.2 * m)
                    xs.append(xlj)
                    ts.append(act * attv[j])
                if H == 4:
                    l0 = (dsc & 3) * 4
                    den = zero
                    aes = []
                    for hh in range(4):
                        alpha = jnp.minimum(
                            jnp.sum(ts[2 * hh] + ts[2 * hh + 1]), 75.0)
                        aev = jnp.exp(jnp.full((16,), alpha, jnp.float32))
                        aes.append(aev)
                        den = jnp.where(lane == l0 + hh, aev, den)
                    for j in range(8):
                        xl_r[p, e, pl.ds(j * 16, 16)] = xs[j] * aes[j // 2]
                    den_st[p, e, pl.ds(0, 16)] = den
                else:
                    l0 = dsc & 15
                    s = ts[0]
                    for j in range(1, 8):
                        s = s + ts[j]
                    alpha = jnp.minimum(jnp.sum(s), 75.0)
                    aev = jnp.exp(jnp.full((16,), alpha, jnp.float32))
                    for j in range(8):
                        xl_r[p, e, pl.ds(j * 16, 16)] = xs[j] * aev
                    den_st[p, e, pl.ds(0, 16)] = jnp.where(lane == l0, aev,
                                                           zero)

        start_idx(0, 0)
        start_idx(1, 1)
        start_idx(2, 2)
        wait_idx(0)
        start_gather(0, 0)

        def group(g, _):
            for c in range(4):
                k = 4 * g + c
                r = c
                p = c % 2

                @pl.when(k >= 1)
                def _():
                    wait_scatter(1 - p)

                @pl.when(k + 1 < NCH)
                def _():
                    wait_idx((c + 1) % 4)
                    start_gather((c + 1) % 4, 1 - p)
                wait_gather(p)
                compute(r, p)
                start_scatter(r, p)

                @pl.when(k + 3 < NCH)
                def _():
                    start_idx(k + 3, (c + 3) % 4)
            return 0

        lax.fori_loop(0, NCH // 4, group, 0, unroll=False)
        wait_scatter(1)
        plsc.subcore_barrier()

        for i in range((NRCH + 15) // 16):
            k = i * 16 + sid

            @pl.when(k < NRCH)
            def _():
                pltpu.sync_copy(acc_sh.at[pl.ds(k * CHB, CHB)], xl_r.at[0])
                pltpu.sync_copy(xl_r.at[0],
                                out_hbm.at[cid, pl.ds(k * CHB, CHB)])
        for i in range((DFULL + 16) // 16):
            k = i * 16 + sid

            @pl.when(k < DFULL)
            def _():
                pltpu.sync_copy(den_sh.at[pl.ds(k * CHB, CHB)], den_st.at[0])
                pltpu.sync_copy(den_st.at[0],
                                dout_hbm.at[cid, pl.ds(k * CHB, CHB)])
            if DREM:
                @pl.when(k == DFULL)
                def _():
                    pltpu.sync_copy(den_sh.at[pl.ds(DFULL * CHB, DREM)],
                                    den_st.at[0, pl.ds(0, DREM)])
                    pltpu.sync_copy(den_st.at[0, pl.ds(0, DREM)],
                                    dout_hbm.at[cid, pl.ds(DFULL * CHB, DREM)])

    return sc_edge


_sc_edge_h4 = _make_sc_edge(4)
_sc_edge_h1 = _make_sc_edge(1)


def _combine_bn(acc_ref, den_ref, gamma_ref, beta_ref, cbias_ref, H, final):
    num = acc_ref[0] + acc_ref[1]
    den = den_ref[0] + den_ref[1]
    row = lax.broadcasted_iota(jnp.int32, (H, HID), 0)
    col = lax.broadcasted_iota(jnp.int32, (H, HID), 1)
    if H == 4:
        R = _f32(row == col // 32)
    else:
        R = jnp.ones((1, HID), jnp.float32) + 0.0 * _f32(row + col)
    denf = jnp.dot(den, R, preferred_element_type=jnp.float32, precision=lax.Precision.HIGHEST)
    o = num / (denf + 1e-16) + cbias_ref[...]
    mu = jnp.sum(o, axis=0, keepdims=True) * (1.0 / NN)
    d = o - mu
    var = jnp.sum(d * d, axis=0, keepdims=True) * (1.0 / NN)
    o = d * lax.rsqrt(var + 1e-5) * gamma_ref[...] + beta_ref[...]
    if not final:
        o = jnp.where(o >= 0.0, o, 0.01 * o)
    return o


def _make_tc_mid(H):
    def body(acc_ref, den_ref, gamma_ref, beta_ref, cbias_ref, Wl_ref, bl_ref,
             Wr_ref, br_ref, We_ref, tab_ref,
             xl_out, xr_out, eptab_out):
        o = _combine_bn(acc_ref, den_ref, gamma_ref, beta_ref, cbias_ref, H,
                        False)
        xl_out[...] = (jnp.dot(o, Wl_ref[...],
                               preferred_element_type=jnp.float32)
                       + bl_ref[...])
        xr_out[...] = (jnp.dot(o, Wr_ref[...],
                               preferred_element_type=jnp.float32)
                       + br_ref[...])
        eptab_out[...] = jnp.dot(tab_ref[...], We_ref[...],
                                 preferred_element_type=jnp.float32)

    def run(acc, den, gamma, beta, cbias, Wl, bl, Wr, br, We, tab):
        return pl.pallas_call(
            body,
            out_shape=(
                jax.ShapeDtypeStruct((NN, HID), jnp.float32),
                jax.ShapeDtypeStruct((NN, HID), jnp.float32),
                jax.ShapeDtypeStruct((TAB, HID), jnp.float32),
            ),
        )(acc, den, gamma, beta, cbias, Wl, bl, Wr, br, We, tab)

    return run


_tc_mid_h4 = _make_tc_mid(4)
_tc_mid_h1 = _make_tc_mid(1)


def _tc_final_body(acc_ref, den_ref, gamma_ref, beta_ref, cbias_ref,
                   batch_ref, W1_ref, b1_ref, W2_ref, b2_ref, W3_ref, b3_ref,
                   W4_ref, b4_ref, out_ref):
    o = _combine_bn(acc_ref, den_ref, gamma_ref, beta_ref, cbias_ref, 1, True)
    oh = _f32(batch_ref[...] == lax.broadcasted_iota(jnp.int32, (NN, GG), 1))
    dnum = (((0,), (0,)), ((), ()))
    sums = lax.dot_general(oh, o, dnum,
                           preferred_element_type=jnp.float32, precision=lax.Precision.HIGHEST)
    cnt = jnp.sum(oh, axis=0, keepdims=True)
    g = sums / jnp.maximum(cnt.reshape(GG, 1), 1.0)
    g = jnp.maximum(jnp.dot(g, W1_ref[...],
                            preferred_element_type=jnp.float32) + b1_ref[...],
                    0.0)
    g = jnp.maximum(jnp.dot(g, W2_ref[...],
                            preferred_element_type=jnp.float32) + b2_ref[...],
                    0.0)
    g = jnp.maximum(jnp.dot(g, W3_ref[...],
                            preferred_element_type=jnp.float32) + b3_ref[...],
                    0.0)
    out_ref[...] = (jnp.dot(g, W4_ref[...],
                            preferred_element_type=jnp.float32) + b4_ref[...])


def _tc_final(acc, den, gamma, beta, cbias, batch, W1, b1, W2, b2, W3, b3,
              W4, b4):
    return pl.pallas_call(
        _tc_final_body,
        out_shape=jax.ShapeDtypeStruct((GG, NCLS), jnp.float32),
    )(acc, den, gamma, beta, cbias, batch, W1, b1, W2, b2, W3, b3, W4, b4)


def kernel(x, edge_index, edge_attr, batch, atom_emb_W, atom_lin_W,
           atom_lin_b, edge_emb_W, edge_lin_W, edge_lin_b, lin_l_W, lin_l_b,
           lin_r_W, lin_r_b, lin_edge_W, att, conv_bias, bn_gamma, bn_beta,
           W1, b1, W2, b2, W3, b3, W4, b4):
    src = edge_index[0].reshape(NWORK, NCH, CHB)
    dst = edge_index[1].reshape(NWORK, NCH, CHB)
    ea0 = edge_attr[:, 0].reshape(EE // 128, 128)
    ea1 = edge_attr[:, 1].reshape(EE // 128, 128)

    xl, xr, eptab, tab, eid = _tc0(
        x, ea0, ea1, atom_emb_W, atom_lin_W, atom_lin_b.reshape(1, -1),
        edge_emb_W, edge_lin_W, edge_lin_b.reshape(1, -1),
        lin_l_W[0], lin_l_b[0].reshape(1, -1),
        lin_r_W[0], lin_r_b[0].reshape(1, -1), lin_edge_W[0])
    eid = eid.reshape(NWORK, NCH, CHB)

    acc, dacc = _sc_edge_h4(xl, xr, eptab, src, dst, eid, att[0])
    xl, xr, eptab = _tc_mid_h4(
        acc, dacc.reshape(2, NN, 4),
        bn_gamma[0].reshape(1, -1), bn_beta[0].reshape(1, -1),
        conv_bias[0].reshape(1, -1),
        lin_l_W[1], lin_l_b[1].reshape(1, -1),
        lin_r_W[1], lin_r_b[1].reshape(1, -1), lin_edge_W[1], tab)

    acc, dacc = _sc_edge_h1(xl, xr, eptab, src, dst, eid, att[1])
    xl, xr, eptab = _tc_mid_h1(
        acc, dacc.reshape(2, NN, 1),
        bn_gamma[1].reshape(1, -1), bn_beta[1].reshape(1, -1),
        conv_bias[1].reshape(1, -1),
        lin_l_W[2], lin_l_b[2].reshape(1, -1),
        lin_r_W[2], lin_r_b[2].reshape(1, -1), lin_edge_W[2], tab)

    acc, dacc = _sc_edge_h1(xl, xr, eptab, src, dst, eid, att[2])
    out = _tc_final(
        acc, dacc.reshape(2, NN, 1),
        bn_gamma[2].reshape(1, -1), bn_beta[2].reshape(1, -1),
        conv_bias[2].reshape(1, -1), batch.reshape(NN, 1),
        W1, b1.reshape(1, -1), W2, b2.reshape(1, -1),
        W3, b3.reshape(1, -1), W4, b4.reshape(1, -1))
    return out
